# baseline (speedup 1.0000x reference)
import jax
import jax.numpy as jnp
from jax.experimental import pallas as pl
from jax.experimental.pallas import tpu as pltpu

AUG_T = 1024
B, L, D = 16, 4096, 128
CH = 512
CR = 1
NC = B // CR
NBUF = 16


def _body(lens_ref, x_ref, o_ref, buf, in_sem, out_sem):
    def start_in(k):
        pltpu.make_async_copy(
            x_ref.at[pl.ds(k * CR, CR)], buf.at[k % NBUF], in_sem.at[k % NBUF]
        ).start()

    def wait_in(k):
        pltpu.make_async_copy(
            x_ref.at[pl.ds(k * CR, CR)], buf.at[k % NBUF], in_sem.at[k % NBUF]
        ).wait()

    def start_out(k):
        pltpu.make_async_copy(
            buf.at[k % NBUF], o_ref.at[pl.ds(k * CR, CR)], out_sem.at[k % NBUF]
        ).start()

    def wait_out(k):
        pltpu.make_async_copy(
            buf.at[k % NBUF], o_ref.at[pl.ds(k * CR, CR)], out_sem.at[k % NBUF]
        ).wait()

    def zero_chunk(k):
        s = k % NBUF
        for r in range(CR):
            slen = lens_ref[k * CR + r]
            is_long = slen > AUG_T
            for c in range(L // CH):
                base = c * CH

                @pl.when(is_long & (base < slen))
                def _mask(s=s, r=r, base=base, slen=slen):
                    first = -(-base // 10) * 10
                    for p in range(first, base + CH, 10):
                        buf[s, r, pl.ds(p, 1), :] = jnp.where(
                            p < slen, 0.0, buf[s, r, pl.ds(p, 1), :]
                        )

    for k in range(min(NBUF, NC)):
        start_in(k)
    for k in range(NC):
        if k + 1 < NC and k + 1 >= NBUF:
            wait_out(k + 1 - NBUF)
            start_in(k + 1)
        wait_in(k)
        zero_chunk(k)
        start_out(k)
    for k in range(max(0, NC - NBUF), NC):
        wait_out(k)


def kernel(sequences, seq_lens):
    out = pl.pallas_call(
        _body,
        in_specs=[
            pl.BlockSpec(memory_space=pltpu.SMEM),
            pl.BlockSpec(memory_space=pl.MemorySpace.ANY),
        ],
        out_specs=pl.BlockSpec(memory_space=pl.MemorySpace.ANY),
        out_shape=jax.ShapeDtypeStruct((B, L, D), jnp.float32),
        scratch_shapes=[
            pltpu.VMEM((NBUF, CR, L, D), jnp.float32),
            pltpu.SemaphoreType.DMA((NBUF,)),
            pltpu.SemaphoreType.DMA((NBUF,)),
        ],
    )(seq_lens, sequences)
    return out, seq_lens


# speedup vs baseline: 1.0146x; 1.0146x over previous
import jax
import jax.numpy as jnp
from jax.experimental import pallas as pl
from jax.experimental.pallas import tpu as pltpu

AUG_T = 1024
B, L, D = 16, 4096, 128
CH = 512
CR = 4
NC = B // CR
NBUF = 4


def _body(lens_ref, x_ref, o_ref, buf, in_sem, out_sem):
    def start_in(k):
        pltpu.make_async_copy(
            x_ref.at[pl.ds(k * CR, CR)], buf.at[k % NBUF], in_sem.at[k % NBUF]
        ).start()

    def wait_in(k):
        pltpu.make_async_copy(
            x_ref.at[pl.ds(k * CR, CR)], buf.at[k % NBUF], in_sem.at[k % NBUF]
        ).wait()

    def start_out(k):
        pltpu.make_async_copy(
            buf.at[k % NBUF], o_ref.at[pl.ds(k * CR, CR)], out_sem.at[k % NBUF]
        ).start()

    def wait_out(k):
        pltpu.make_async_copy(
            buf.at[k % NBUF], o_ref.at[pl.ds(k * CR, CR)], out_sem.at[k % NBUF]
        ).wait()

    def zero_chunk(k):
        s = k % NBUF
        for r in range(CR):
            slen = lens_ref[k * CR + r]
            is_long = slen > AUG_T
            for c in range(L // CH):
                base = c * CH

                @pl.when(is_long & (base < slen))
                def _mask(s=s, r=r, base=base, slen=slen):
                    first = -(-base // 10) * 10
                    for p in range(first, base + CH, 10):
                        buf[s, r, pl.ds(p, 1), :] = jnp.where(
                            p < slen, 0.0, buf[s, r, pl.ds(p, 1), :]
                        )

    for k in range(min(NBUF, NC)):
        start_in(k)
    for k in range(NC):
        if k + 1 < NC and k + 1 >= NBUF:
            wait_out(k + 1 - NBUF)
            start_in(k + 1)
        wait_in(k)
        zero_chunk(k)
        start_out(k)
    for k in range(max(0, NC - NBUF), NC):
        wait_out(k)


def kernel(sequences, seq_lens):
    out = pl.pallas_call(
        _body,
        in_specs=[
            pl.BlockSpec(memory_space=pltpu.SMEM),
            pl.BlockSpec(memory_space=pl.MemorySpace.ANY),
        ],
        out_specs=pl.BlockSpec(memory_space=pl.MemorySpace.ANY),
        out_shape=jax.ShapeDtypeStruct((B, L, D), jnp.float32),
        scratch_shapes=[
            pltpu.VMEM((NBUF, CR, L, D), jnp.float32),
            pltpu.SemaphoreType.DMA((NBUF,)),
            pltpu.SemaphoreType.DMA((NBUF,)),
        ],
    )(seq_lens, sequences)
    return out, seq_lens
